# baseline (speedup 1.0000x reference)
import jax
import jax.numpy as jnp
from jax import lax
from jax.experimental import pallas as pl
from jax.experimental.pallas import tpu as pltpu
from jax.experimental.pallas import tpu_sc as plsc

B = 1024
S = 512
HID = 128

NC = 2
NS = 16
NW = NC * NS
TOK = B * S
CH = 128

K = 8
BC = B // K
TOK_C = TOK // K
PER_WC = TOK_C // NW
NCH_C = PER_WC // CH


HID2 = HID // 2
NSLOT = 4
_HI_MASK = -65536


def _sc_gather_body(table_hbm, ids_hbm, out_hbm, idx_v, rows_v, wout_v,
                    gsems, ssems):
    wid = lax.axis_index("s") * NC + lax.axis_index("c")
    base = wid * PER_WC
    pltpu.sync_copy(ids_hbm.at[wid], idx_v)

    iota = lax.iota(jnp.int32, 16)
    idx_e = lax.bitwise_and(iota + iota, 15)[:, None]
    idx_o = lax.bitwise_and(iota + iota + 1, 15)[:, None]
    lane_lo = iota < 8
    dnums = lax.GatherDimensionNumbers(
        offset_dims=(), collapsed_slice_dims=(0,), start_index_map=(0,))

    def perm(v, idx):
        return lax.gather(v, idx, dnums, (1,),
                          mode=lax.GatherScatterMode.PROMISE_IN_BOUNDS)

    def start_gather(j, slot):
        pltpu.async_copy(
            table_hbm.at[idx_v.at[j]], rows_v.at[slot], gsems[slot])

    def wait_gather(slot):
        pltpu.make_async_copy(
            table_hbm.at[idx_v.at[slot]], rows_v.at[slot], gsems[slot]).wait()

    def sdesc(j, slot):
        return pltpu.make_async_copy(
            wout_v.at[slot],
            out_hbm.at[pl.ds((base + j * CH) * HID2, CH * HID2)],
            ssems[slot])

    def convert(slot):
        def row(t, carry):
            for w in range(HID // 32):
                lo = rows_v[slot, t, pl.ds(w * 32, 16)]
                hi = rows_v[slot, t, pl.ds(w * 32 + 16, 16)]
                ev = jnp.where(lane_lo, perm(lo, idx_e), perm(hi, idx_e))
                od = jnp.where(lane_lo, perm(lo, idx_o), perm(hi, idx_o))
                be = lax.bitcast_convert_type(ev, jnp.int32)
                bo = lax.bitcast_convert_type(od, jnp.int32)
                word = lax.bitwise_or(
                    lax.shift_right_logical(be, 16),
                    lax.bitwise_and(bo, _HI_MASK))
                wout_v[slot, pl.ds(t * HID2 + w * 16, 16)] = word
            return carry

        lax.fori_loop(0, CH, row, 0)

    start_gather(0, 0)
    start_gather(1, 1)

    def outer(m, carry):
        for p in range(NSLOT):
            c = m * NSLOT + p
            wait_gather(p)
            convert(p)
            sdesc(c, p).start()
            p2 = (p + 2) % NSLOT

            @pl.when(c >= 2)
            def _wait_old_scatter():
                sdesc(c - 2, p2).wait()

            @pl.when(c + 2 < NCH_C)
            def _start_next_gather():
                start_gather(c + 2, p2)
        return carry

    lax.fori_loop(0, NCH_C // NSLOT, outer, 0)
    sdesc(NCH_C - 2, (NCH_C - 2) % NSLOT).wait()
    sdesc(NCH_C - 1, (NCH_C - 1) % NSLOT).wait()


def _sc_gather(token_table, ids3):
    mesh = plsc.VectorSubcoreMesh(core_axis_name="c", subcore_axis_name="s")
    return pl.kernel(
        _sc_gather_body,
        out_type=jax.ShapeDtypeStruct((TOK_C * HID2,), jnp.int32),
        mesh=mesh,
        scratch_types=[
            pltpu.VMEM((NCH_C, CH), jnp.int32),
            pltpu.VMEM((NSLOT, CH, HID), jnp.float32),
            pltpu.VMEM((NSLOT, CH * HID2), jnp.int32),
            [pltpu.SemaphoreType.DMA] * NSLOT,
            [pltpu.SemaphoreType.DMA] * NSLOT,
        ],
    )(token_table, ids3)


RB = 8


def _ln_body(x_ref, tt_ref, pos_ref, ty_ref, w_ref, b_ref, prev_ref, o_ref):
    del prev_ref
    x = x_ref[...].astype(jnp.float32)
    tt = tt_ref[...].astype(jnp.float32)
    pos = pos_ref[...]
    t0 = ty_ref[0]
    dt = ty_ref[1] - t0
    e = x + pos[None, :, :] + t0[None, None, :] + tt[:, :, None] * dt[None, None, :]
    mean = jnp.mean(e, axis=-1, keepdims=True)
    var = jnp.mean(jnp.square(e - mean), axis=-1, keepdims=True)
    normed = (e - mean) * lax.rsqrt(var + 1e-5)
    o_ref[...] = normed * w_ref[0][None, None, :] + b_ref[0][None, None, :]


def _ln_body_first(x_ref, tt_ref, pos_ref, ty_ref, w_ref, b_ref, o_ref):
    _ln_body(x_ref, tt_ref, pos_ref, ty_ref, w_ref, b_ref, None, o_ref)


def _tc_ln_chunk(c, x, tt_c, pos_enc, type_table, w2, b2, prev):
    specs = [
        pl.BlockSpec((RB, S, HID), lambda i: (i, 0, 0)),
        pl.BlockSpec((RB, S), lambda i: (i, 0)),
        pl.BlockSpec((S, HID), lambda i: (0, 0)),
        pl.BlockSpec((2, HID), lambda i: (0, 0)),
        pl.BlockSpec((1, HID), lambda i: (0, 0)),
        pl.BlockSpec((1, HID), lambda i: (0, 0)),
    ]
    args = [x, tt_c, pos_enc, type_table, w2, b2]
    body = _ln_body_first
    aliases = {}
    if prev is not None:
        specs.append(pl.BlockSpec(memory_space=pltpu.MemorySpace.HBM))
        args.append(prev)
        body = _ln_body
        aliases = {6: 0}
    return pl.pallas_call(
        body,
        grid=(BC // RB,),
        in_specs=specs,
        out_specs=pl.BlockSpec((RB, S, HID), lambda i, _c=c: (_c * (BC // RB) + i, 0, 0)),
        out_shape=jax.ShapeDtypeStruct((B, S, HID), jnp.float32),
        input_output_aliases=aliases,
    )(*args)


def kernel(input_ids, token_type_ids, token_table, type_table, pos_enc, ln_weight, ln_bias):
    ids4 = input_ids.astype(jnp.int32).reshape(K, NW, NCH_C, CH)
    tt4 = token_type_ids.reshape(K, BC, S)
    w2 = ln_weight.reshape(1, HID)
    b2 = ln_bias.reshape(1, HID)
    gathered = []
    for c in range(K):
        g = _sc_gather(token_table, ids4[c])
        gbf = lax.bitcast_convert_type(g, jnp.bfloat16)
        gathered.append(gbf.reshape(BC, S, HID))
    out = None
    for c in range(K):
        out = _tc_ln_chunk(c, gathered[c], tt4[c], pos_enc, type_table, w2, b2, out)
    return out


# speedup vs baseline: 3.2282x; 3.2282x over previous
import jax
import jax.numpy as jnp
from jax import lax
from jax.experimental import pallas as pl
from jax.experimental.pallas import tpu as pltpu
from jax.experimental.pallas import tpu_sc as plsc

B = 1024
S = 512
HID = 128

NC = 2
NS = 16
NW = NC * NS
TOK = B * S
CH = 128

K = 8
BC = B // K
TOK_C = TOK // K
PER_WC = TOK_C // NW
NCH_C = PER_WC // CH


def _sc_gather_body(table_hbm, ids_hbm, out_hbm, idx_v, rows_v, gsem):
    wid = lax.axis_index("s") * NC + lax.axis_index("c")
    base = wid * PER_WC
    pltpu.sync_copy(ids_hbm.at[wid], idx_v)

    def step(j, carry):
        pltpu.async_copy(table_hbm.at[idx_v.at[j]], rows_v, gsem).wait()
        pltpu.sync_copy(rows_v, out_hbm.at[pl.ds(base + j * CH, CH)])
        return carry

    lax.fori_loop(0, NCH_C, step, 0)


def _sc_gather(token_table, ids3):
    mesh = plsc.VectorSubcoreMesh(core_axis_name="c", subcore_axis_name="s")
    return pl.kernel(
        _sc_gather_body,
        out_type=jax.ShapeDtypeStruct((TOK_C, HID), jnp.float32),
        mesh=mesh,
        scratch_types=[
            pltpu.VMEM((NCH_C, CH), jnp.int32),
            pltpu.VMEM((CH, HID), jnp.float32),
            pltpu.SemaphoreType.DMA,
        ],
    )(token_table, ids3)


RB = 8


def _ln_body(x_ref, tt_ref, pos_ref, ty_ref, w_ref, b_ref, prev_ref, o_ref):
    del prev_ref
    x = x_ref[...]
    tt = tt_ref[...].astype(jnp.float32)
    pos = pos_ref[...]
    t0 = ty_ref[0]
    dt = ty_ref[1] - t0
    e = x + pos[None, :, :] + t0[None, None, :] + tt[:, :, None] * dt[None, None, :]
    mean = jnp.mean(e, axis=-1, keepdims=True)
    var = jnp.mean(jnp.square(e - mean), axis=-1, keepdims=True)
    normed = (e - mean) * lax.rsqrt(var + 1e-5)
    o_ref[...] = normed * w_ref[0][None, None, :] + b_ref[0][None, None, :]


def _ln_body_first(x_ref, tt_ref, pos_ref, ty_ref, w_ref, b_ref, o_ref):
    _ln_body(x_ref, tt_ref, pos_ref, ty_ref, w_ref, b_ref, None, o_ref)


def _tc_ln_chunk(c, x, tt_c, pos_enc, type_table, w2, b2, prev):
    specs = [
        pl.BlockSpec((RB, S, HID), lambda i: (i, 0, 0)),
        pl.BlockSpec((RB, S), lambda i: (i, 0)),
        pl.BlockSpec((S, HID), lambda i: (0, 0)),
        pl.BlockSpec((2, HID), lambda i: (0, 0)),
        pl.BlockSpec((1, HID), lambda i: (0, 0)),
        pl.BlockSpec((1, HID), lambda i: (0, 0)),
    ]
    args = [x, tt_c, pos_enc, type_table, w2, b2]
    body = _ln_body_first
    aliases = {}
    if prev is not None:
        specs.append(pl.BlockSpec(memory_space=pltpu.MemorySpace.HBM))
        args.append(prev)
        body = _ln_body
        aliases = {6: 0}
    return pl.pallas_call(
        body,
        grid=(BC // RB,),
        in_specs=specs,
        out_specs=pl.BlockSpec((RB, S, HID), lambda i, _c=c: (_c * (BC // RB) + i, 0, 0)),
        out_shape=jax.ShapeDtypeStruct((B, S, HID), jnp.float32),
        input_output_aliases=aliases,
    )(*args)


def kernel(input_ids, token_type_ids, token_table, type_table, pos_enc, ln_weight, ln_bias):
    ids4 = input_ids.astype(jnp.int32).reshape(K, NW, NCH_C, CH)
    tt4 = token_type_ids.reshape(K, BC, S)
    w2 = ln_weight.reshape(1, HID)
    b2 = ln_bias.reshape(1, HID)
    gathered = [_sc_gather(token_table, ids4[c]).reshape(BC, S, HID) for c in range(K)]
    out = None
    for c in range(K):
        out = _tc_ln_chunk(c, gathered[c], tt4[c], pos_enc, type_table, w2, b2, out)
    return out


# speedup vs baseline: 3.3943x; 1.0515x over previous
import jax
import jax.numpy as jnp
from jax import lax
from jax.experimental import pallas as pl
from jax.experimental.pallas import tpu as pltpu
from jax.experimental.pallas import tpu_sc as plsc

B = 1024
S = 512
HID = 128

NC = 2
NS = 16
NW = NC * NS
TOK = B * S
CH = 128

K = 8
BC = B // K
TOK_C = TOK // K
PER_WC = TOK_C // NW
NCH_C = PER_WC // CH


NSLOT = 4


def _sc_gather_body(table_hbm, ids_hbm, out_hbm, idx_v, rows_v, gsems, ssems):
    wid = lax.axis_index("s") * NC + lax.axis_index("c")
    base = wid * PER_WC
    pltpu.sync_copy(ids_hbm.at[wid], idx_v)

    def start_gather(j, slot):
        pltpu.async_copy(
            table_hbm.at[idx_v.at[j]], rows_v.at[slot], gsems[slot])

    def wait_gather(slot):
        pltpu.make_async_copy(
            table_hbm.at[idx_v.at[slot]], rows_v.at[slot], gsems[slot]).wait()

    def sdesc(j, slot):
        return pltpu.make_async_copy(
            rows_v.at[slot],
            out_hbm.at[pl.ds(base + j * CH, CH)],
            ssems[slot])

    start_gather(0, 0)
    start_gather(1, 1)

    def outer(m, carry):
        for p in range(NSLOT):
            c = m * NSLOT + p
            wait_gather(p)
            sdesc(c, p).start()
            p2 = (p + 2) % NSLOT

            @pl.when(c >= 2)
            def _wait_old_scatter():
                sdesc(c - 2, p2).wait()

            @pl.when(c + 2 < NCH_C)
            def _start_next_gather():
                start_gather(c + 2, p2)
        return carry

    lax.fori_loop(0, NCH_C // NSLOT, outer, 0)
    sdesc(NCH_C - 2, (NCH_C - 2) % NSLOT).wait()
    sdesc(NCH_C - 1, (NCH_C - 1) % NSLOT).wait()


def _sc_gather(token_table, ids3):
    mesh = plsc.VectorSubcoreMesh(core_axis_name="c", subcore_axis_name="s")
    return pl.kernel(
        _sc_gather_body,
        out_type=jax.ShapeDtypeStruct((TOK_C, HID), jnp.float32),
        mesh=mesh,
        scratch_types=[
            pltpu.VMEM((NCH_C, CH), jnp.int32),
            pltpu.VMEM((NSLOT, CH, HID), jnp.float32),
            [pltpu.SemaphoreType.DMA] * NSLOT,
            [pltpu.SemaphoreType.DMA] * NSLOT,
        ],
    )(token_table, ids3)


RB = 8


def _ln_body(x_ref, tt_ref, pos_ref, ty_ref, w_ref, b_ref, prev_ref, o_ref):
    del prev_ref
    x = x_ref[...]
    tt = tt_ref[...].astype(jnp.float32)
    pos = pos_ref[...]
    t0 = ty_ref[0]
    dt = ty_ref[1] - t0
    e = x + pos[None, :, :] + t0[None, None, :] + tt[:, :, None] * dt[None, None, :]
    mean = jnp.mean(e, axis=-1, keepdims=True)
    var = jnp.mean(jnp.square(e - mean), axis=-1, keepdims=True)
    normed = (e - mean) * lax.rsqrt(var + 1e-5)
    o_ref[...] = normed * w_ref[0][None, None, :] + b_ref[0][None, None, :]


def _ln_body_first(x_ref, tt_ref, pos_ref, ty_ref, w_ref, b_ref, o_ref):
    _ln_body(x_ref, tt_ref, pos_ref, ty_ref, w_ref, b_ref, None, o_ref)


def _tc_ln_chunk(c, x, tt_c, pos_enc, type_table, w2, b2, prev):
    specs = [
        pl.BlockSpec((RB, S, HID), lambda i: (i, 0, 0)),
        pl.BlockSpec((RB, S), lambda i: (i, 0)),
        pl.BlockSpec((S, HID), lambda i: (0, 0)),
        pl.BlockSpec((2, HID), lambda i: (0, 0)),
        pl.BlockSpec((1, HID), lambda i: (0, 0)),
        pl.BlockSpec((1, HID), lambda i: (0, 0)),
    ]
    args = [x, tt_c, pos_enc, type_table, w2, b2]
    body = _ln_body_first
    aliases = {}
    if prev is not None:
        specs.append(pl.BlockSpec(memory_space=pltpu.MemorySpace.HBM))
        args.append(prev)
        body = _ln_body
        aliases = {6: 0}
    return pl.pallas_call(
        body,
        grid=(BC // RB,),
        in_specs=specs,
        out_specs=pl.BlockSpec((RB, S, HID), lambda i, _c=c: (_c * (BC // RB) + i, 0, 0)),
        out_shape=jax.ShapeDtypeStruct((B, S, HID), jnp.float32),
        input_output_aliases=aliases,
    )(*args)


def kernel(input_ids, token_type_ids, token_table, type_table, pos_enc, ln_weight, ln_bias):
    ids4 = input_ids.astype(jnp.int32).reshape(K, NW, NCH_C, CH)
    tt4 = token_type_ids.reshape(K, BC, S)
    w2 = ln_weight.reshape(1, HID)
    b2 = ln_bias.reshape(1, HID)
    gathered = [_sc_gather(token_table, ids4[c]).reshape(BC, S, HID) for c in range(K)]
    out = None
    for c in range(K):
        out = _tc_ln_chunk(c, gathered[c], tt4[c], pos_enc, type_table, w2, b2, out)
    return out


# speedup vs baseline: 3.5622x; 1.0495x over previous
import jax
import jax.numpy as jnp
from jax import lax
from jax.experimental import pallas as pl
from jax.experimental.pallas import tpu as pltpu
from jax.experimental.pallas import tpu_sc as plsc

B = 1024
S = 512
HID = 128

NC = 2
NS = 16
NW = NC * NS
TOK = B * S
CH = 128

K = 8
BC = B // K
TOK_C = TOK // K
PER_WC = TOK_C // NW
NCH_C = PER_WC // CH


NSLOT = 4


def _sc_gather_body(table_hbm, ids_hbm, out_hbm, idx_v, rows_v, gsems, ssems):
    wid = lax.axis_index("s") * NC + lax.axis_index("c")
    base = wid * PER_WC
    pltpu.sync_copy(ids_hbm.at[wid], idx_v)

    def start_gather(j, slot):
        pltpu.async_copy(
            table_hbm.at[idx_v.at[j]], rows_v.at[slot], gsems[slot])

    def wait_gather(slot):
        pltpu.make_async_copy(
            table_hbm.at[idx_v.at[slot]], rows_v.at[slot], gsems[slot]).wait()

    def sdesc(j, slot):
        return pltpu.make_async_copy(
            rows_v.at[slot],
            out_hbm.at[pl.ds(base + j * CH, CH)],
            ssems[slot])

    start_gather(0, 0)
    start_gather(1, 1)

    def outer(m, carry):
        for p in range(NSLOT):
            c = m * NSLOT + p
            wait_gather(p)
            sdesc(c, p).start()
            p2 = (p + 2) % NSLOT

            @pl.when(c >= 2)
            def _wait_old_scatter():
                sdesc(c - 2, p2).wait()

            @pl.when(c + 2 < NCH_C)
            def _start_next_gather():
                start_gather(c + 2, p2)
        return carry

    lax.fori_loop(0, NCH_C // NSLOT, outer, 0)
    sdesc(NCH_C - 2, (NCH_C - 2) % NSLOT).wait()
    sdesc(NCH_C - 1, (NCH_C - 1) % NSLOT).wait()


def _sc_gather(token_table, ids3):
    mesh = plsc.VectorSubcoreMesh(core_axis_name="c", subcore_axis_name="s")
    return pl.kernel(
        _sc_gather_body,
        out_type=jax.ShapeDtypeStruct((TOK_C, HID), jnp.float32),
        mesh=mesh,
        scratch_types=[
            pltpu.VMEM((NCH_C, CH), jnp.int32),
            pltpu.VMEM((NSLOT, CH, HID), jnp.float32),
            [pltpu.SemaphoreType.DMA] * NSLOT,
            [pltpu.SemaphoreType.DMA] * NSLOT,
        ],
    )(token_table, ids3)


RB = 16


def _ln_body(x_ref, tt_ref, pos_ref, ty_ref, w_ref, b_ref, prev_ref, o_ref):
    del prev_ref
    x = x_ref[...]
    tt = tt_ref[...].astype(jnp.float32)
    pos = pos_ref[...]
    t0 = ty_ref[0]
    dt = ty_ref[1] - t0
    e = x + pos[None, :, :] + t0[None, None, :] + tt[:, :, None] * dt[None, None, :]
    mean = jnp.mean(e, axis=-1, keepdims=True)
    var = jnp.mean(jnp.square(e - mean), axis=-1, keepdims=True)
    normed = (e - mean) * lax.rsqrt(var + 1e-5)
    o_ref[...] = normed * w_ref[0][None, None, :] + b_ref[0][None, None, :]


def _ln_body_first(x_ref, tt_ref, pos_ref, ty_ref, w_ref, b_ref, o_ref):
    _ln_body(x_ref, tt_ref, pos_ref, ty_ref, w_ref, b_ref, None, o_ref)


def _tc_ln_chunk(c, x, tt_c, pos_enc, type_table, w2, b2, prev):
    specs = [
        pl.BlockSpec((RB, S, HID), lambda i: (i, 0, 0)),
        pl.BlockSpec((RB, S), lambda i: (i, 0)),
        pl.BlockSpec((S, HID), lambda i: (0, 0)),
        pl.BlockSpec((2, HID), lambda i: (0, 0)),
        pl.BlockSpec((1, HID), lambda i: (0, 0)),
        pl.BlockSpec((1, HID), lambda i: (0, 0)),
    ]
    args = [x, tt_c, pos_enc, type_table, w2, b2]
    body = _ln_body_first
    aliases = {}
    if prev is not None:
        specs.append(pl.BlockSpec(memory_space=pltpu.MemorySpace.HBM))
        args.append(prev)
        body = _ln_body
        aliases = {6: 0}
    return pl.pallas_call(
        body,
        grid=(BC // RB,),
        in_specs=specs,
        out_specs=pl.BlockSpec((RB, S, HID), lambda i, _c=c: (_c * (BC // RB) + i, 0, 0)),
        out_shape=jax.ShapeDtypeStruct((B, S, HID), jnp.float32),
        input_output_aliases=aliases,
    )(*args)


def kernel(input_ids, token_type_ids, token_table, type_table, pos_enc, ln_weight, ln_bias):
    ids4 = input_ids.astype(jnp.int32).reshape(K, NW, NCH_C, CH)
    tt4 = token_type_ids.reshape(K, BC, S)
    w2 = ln_weight.reshape(1, HID)
    b2 = ln_bias.reshape(1, HID)
    gathered = [_sc_gather(token_table, ids4[c]).reshape(BC, S, HID) for c in range(K)]
    out = None
    for c in range(K):
        out = _tc_ln_chunk(c, gathered[c], tt4[c], pos_enc, type_table, w2, b2, out)
    return out
